# baseline (speedup 1.0000x reference)
import functools

import jax
import jax.numpy as jnp
from jax import lax
from jax.experimental import pallas as pl
from jax.experimental.pallas import tpu as pltpu
from jax.experimental.pallas import tpu_sc as plsc

_NC = 2
_NS = 16
_L = 16
_NCHUNK = 3


def _tables_tc(node_embeds, R, W, b1):
    N, D = node_embeds.shape
    NR = R.shape[0]
    BN = 2048

    def body(ne_ref, r_ref, w_ref, b_ref, ntr_ref, ntc_ref, nth_ref,
             rtr_ref, rtq_ref):
        w = w_ref[...].reshape(5, D)
        nt = lax.dot_general(w, ne_ref[...], (((1,), (1,)), ((), ())),
                             preferred_element_type=jnp.float32)
        rt = lax.dot_general(w, r_ref[...], (((1,), (1,)), ((), ())),
                             preferred_element_type=jnp.float32)
        ntr_ref[...] = nt[0]
        ntc_ref[...] = nt[1]
        nth_ref[...] = nt[3]
        rtr_ref[...] = rt[2]
        rtq_ref[...] = rt[4] + b_ref[0]

    return pl.pallas_call(
        body,
        grid=(pl.cdiv(N, BN),),
        in_specs=[
            pl.BlockSpec((BN, D), lambda i: (i, 0)),
            pl.BlockSpec((NR, D), lambda i: (0, 0)),
            pl.BlockSpec((1, 5 * D), lambda i: (0, 0)),
            pl.BlockSpec((1,), lambda i: (0,)),
        ],
        out_specs=[
            pl.BlockSpec((BN,), lambda i: (i,)),
            pl.BlockSpec((BN,), lambda i: (i,)),
            pl.BlockSpec((BN,), lambda i: (i,)),
            pl.BlockSpec((NR,), lambda i: (0,)),
            pl.BlockSpec((NR,), lambda i: (0,)),
        ],
        out_shape=[
            jax.ShapeDtypeStruct((N,), jnp.float32),
            jax.ShapeDtypeStruct((N,), jnp.float32),
            jax.ShapeDtypeStruct((N,), jnp.float32),
            jax.ShapeDtypeStruct((NR,), jnp.float32),
            jax.ShapeDtypeStruct((NR,), jnp.float32),
        ],
    )(node_embeds, R, W, b1)


def _edge_sum_sc(bei, types, bids, h_index, r_index,
                 ntr, ntc, nth, rtr, rtq):
    E = types.shape[0]
    N = ntr.shape[0]
    NR = rtr.shape[0]
    B = h_index.shape[0]
    NW = _NC * _NS
    assert E % 128 == 0, E
    chunk = (E // 128 // NW) * 128
    tail = E - chunk * NW
    C = chunk // _NCHUNK
    assert chunk % (_NCHUNK * 128) == 0, chunk
    assert tail % _L == 0 and tail <= C, tail
    assert B % _L == 0, B
    mesh = plsc.VectorSubcoreMesh(core_axis_name="c", subcore_axis_name="s")

    @functools.partial(
        pl.kernel,
        mesh=mesh,
        out_type=jax.ShapeDtypeStruct((E,), jnp.float32),
        compiler_params=pltpu.CompilerParams(needs_layout_passes=False),
        scratch_types=[
            pltpu.VMEM((2, 2 * C), jnp.int32),
            pltpu.VMEM((2 * C,), jnp.int32),
            pltpu.VMEM((2 * C,), jnp.int32),
            pltpu.VMEM((N,), jnp.float32),
            pltpu.VMEM((N,), jnp.float32),
            pltpu.VMEM((N,), jnp.float32),
            pltpu.VMEM((NR,), jnp.float32),
            pltpu.VMEM((NR,), jnp.float32),
            pltpu.VMEM((B,), jnp.int32),
            pltpu.VMEM((B,), jnp.int32),
            pltpu.VMEM((B,), jnp.float32),
            pltpu.VMEM((chunk,), jnp.float32),
            pltpu.SemaphoreType.DMA,
            pltpu.SemaphoreType.DMA,
            pltpu.SemaphoreType.DMA,
        ],
    )
    def k(bei_h, types_h, bids_h, hidx_h, ridx_h,
          ntr_h, ntc_h, nth_h, rtr_h, rtq_h, out_h,
          rc_v, types_v, bids_v,
          ntr_v, ntc_v, nth_v, rtr_v, rtq_v,
          hidx_v, ridx_v, H_v, out_v, sem0, sem1, semt):
        wid = lax.axis_index("s") * _NC + lax.axis_index("c")
        base = wid * chunk
        sems = (sem0, sem1)

        def start(lo, n, p):
            return [
                pltpu.async_copy(bei_h.at[:, pl.ds(lo, n)],
                                 rc_v.at[:, pl.ds(p * C, n)], sems[p]),
                pltpu.async_copy(types_h.at[pl.ds(lo, n)],
                                 types_v.at[pl.ds(p * C, n)], sems[p]),
                pltpu.async_copy(bids_h.at[pl.ds(lo, n)],
                                 bids_v.at[pl.ds(p * C, n)], sems[p]),
            ]

        pending = start(base, C, 0)
        tdescs = [
            pltpu.async_copy(ntr_h, ntr_v, semt),
            pltpu.async_copy(ntc_h, ntc_v, semt),
            pltpu.async_copy(nth_h, nth_v, semt),
            pltpu.async_copy(rtr_h, rtr_v, semt),
            pltpu.async_copy(rtq_h, rtq_v, semt),
            pltpu.async_copy(hidx_h, hidx_v, semt),
            pltpu.async_copy(ridx_h, ridx_v, semt),
        ]
        for d in tdescs:
            d.wait()

        for j in range(B // _L):
            hi = hidx_v[pl.ds(j * _L, _L)]
            ri = ridx_v[pl.ds(j * _L, _L)]
            H_v[pl.ds(j * _L, _L)] = (plsc.load_gather(nth_v, [hi])
                                      + plsc.load_gather(rtq_v, [ri]))

        def run_block(p, n, out_lo):
            buf_lo = p * C

            @plsc.parallel_loop(0, n, _L, unroll=16)
            def body(off):
                r = rc_v[0, pl.ds(buf_lo + off, _L)]
                c = rc_v[1, pl.ds(buf_lo + off, _L)]
                t = types_v[pl.ds(buf_lo + off, _L)]
                g = bids_v[pl.ds(buf_lo + off, _L)]
                out_v[pl.ds(out_lo + off, _L)] = (
                    plsc.load_gather(ntr_v, [r])
                    + plsc.load_gather(ntc_v, [c])
                    + plsc.load_gather(rtr_v, [t])
                    + plsc.load_gather(H_v, [g]))

        for ci in range(_NCHUNK):
            p = ci & 1
            nxt = pending if ci + 1 == _NCHUNK else start(
                base + (ci + 1) * C, C, 1 - p)
            for d in pending:
                d.wait()
            pending = nxt
            run_block(p, C, ci * C)

        pltpu.sync_copy(out_v, out_h.at[pl.ds(base, chunk)])

        if tail:
            @pl.when(wid == 0)
            def _():
                tail_lo = NW * chunk
                for d in start(tail_lo, tail, 0):
                    d.wait()
                run_block(0, tail, 0)
                pltpu.sync_copy(out_v.at[pl.ds(0, tail)],
                                out_h.at[pl.ds(tail_lo, tail)])

    return k(bei, types, bids, h_index, r_index,
             ntr, ntc, nth, rtr, rtq)


def kernel(batch_edge_index, batch_edge_type, batch_id, h_index, r_index,
           node_embeds, R, W, b):
    E = batch_edge_type.shape[0]
    ntr, ntc, nth, rtr, rtq = _tables_tc(node_embeds, R, W,
                                         b.astype(jnp.float32))
    out = _edge_sum_sc(
        batch_edge_index, batch_edge_type, batch_id,
        h_index, r_index, ntr, ntc, nth, rtr, rtq,
    )
    return out.reshape(E, 1)


# speedup vs baseline: 1.0332x; 1.0332x over previous
import functools

import jax
import jax.numpy as jnp
from jax import lax
from jax.experimental import pallas as pl
from jax.experimental.pallas import tpu as pltpu
from jax.experimental.pallas import tpu_sc as plsc

_NC = 2
_NS = 16
_L = 16
_NCHUNK = 3


def _tables_tc(node_embeds, R, W, b1):
    N, D = node_embeds.shape
    NR = R.shape[0]

    def body(ne_ref, r_ref, w_ref, b_ref, ntr_ref, ntc_ref, nth_ref,
             rtr_ref, rtq_ref):
        w = w_ref[...].reshape(5, D)
        nt = lax.dot_general(w, ne_ref[...], (((1,), (1,)), ((), ())),
                             preferred_element_type=jnp.float32)
        rt = lax.dot_general(w, r_ref[...], (((1,), (1,)), ((), ())),
                             preferred_element_type=jnp.float32)
        ntr_ref[...] = nt[0]
        ntc_ref[...] = nt[1]
        nth_ref[...] = nt[3]
        rtr_ref[...] = rt[2]
        rtq_ref[...] = rt[4] + b_ref[0]

    return pl.pallas_call(
        body,
        out_shape=[
            jax.ShapeDtypeStruct((N,), jnp.float32),
            jax.ShapeDtypeStruct((N,), jnp.float32),
            jax.ShapeDtypeStruct((N,), jnp.float32),
            jax.ShapeDtypeStruct((NR,), jnp.float32),
            jax.ShapeDtypeStruct((NR,), jnp.float32),
        ],
    )(node_embeds, R, W, b1)


def _edge_sum_sc(bei, types, bids, h_index, r_index,
                 ntr, ntc, nth, rtr, rtq):
    E = types.shape[0]
    N = ntr.shape[0]
    NR = rtr.shape[0]
    B = h_index.shape[0]
    NW = _NC * _NS
    assert E % 128 == 0, E
    chunk = (E // 128 // NW) * 128
    tail = E - chunk * NW
    C = chunk // _NCHUNK
    assert chunk % (_NCHUNK * 128) == 0, chunk
    assert tail % _L == 0 and tail <= C, tail
    assert B % _L == 0, B
    mesh = plsc.VectorSubcoreMesh(core_axis_name="c", subcore_axis_name="s")

    @functools.partial(
        pl.kernel,
        mesh=mesh,
        out_type=jax.ShapeDtypeStruct((E,), jnp.float32),
        compiler_params=pltpu.CompilerParams(needs_layout_passes=False),
        scratch_types=[
            pltpu.VMEM((2, 2 * C), jnp.int32),
            pltpu.VMEM((2 * C,), jnp.int32),
            pltpu.VMEM((2 * C,), jnp.int32),
            pltpu.VMEM((N,), jnp.float32),
            pltpu.VMEM((N,), jnp.float32),
            pltpu.VMEM((N,), jnp.float32),
            pltpu.VMEM((NR,), jnp.float32),
            pltpu.VMEM((NR,), jnp.float32),
            pltpu.VMEM((B,), jnp.int32),
            pltpu.VMEM((B,), jnp.int32),
            pltpu.VMEM((B,), jnp.float32),
            pltpu.VMEM((chunk,), jnp.float32),
            pltpu.SemaphoreType.DMA,
            pltpu.SemaphoreType.DMA,
            pltpu.SemaphoreType.DMA,
            pltpu.SemaphoreType.DMA,
        ],
    )
    def k(bei_h, types_h, bids_h, hidx_h, ridx_h,
          ntr_h, ntc_h, nth_h, rtr_h, rtq_h, out_h,
          rc_v, types_v, bids_v,
          ntr_v, ntc_v, nth_v, rtr_v, rtq_v,
          hidx_v, ridx_v, H_v, out_v, sem0, sem1, semt, semu):
        wid = lax.axis_index("s") * _NC + lax.axis_index("c")
        base = wid * chunk
        sems = (sem0, sem1)

        def start(lo, n, p):
            return [
                pltpu.async_copy(bei_h.at[:, pl.ds(lo, n)],
                                 rc_v.at[:, pl.ds(p * C, n)], sems[p]),
                pltpu.async_copy(types_h.at[pl.ds(lo, n)],
                                 types_v.at[pl.ds(p * C, n)], sems[p]),
                pltpu.async_copy(bids_h.at[pl.ds(lo, n)],
                                 bids_v.at[pl.ds(p * C, n)], sems[p]),
            ]

        pending = start(base, C, 0)
        hdescs = [
            pltpu.async_copy(nth_h, nth_v, semt),
            pltpu.async_copy(rtq_h, rtq_v, semt),
            pltpu.async_copy(hidx_h, hidx_v, semt),
            pltpu.async_copy(ridx_h, ridx_v, semt),
        ]
        tdescs = [
            pltpu.async_copy(ntr_h, ntr_v, semu),
            pltpu.async_copy(ntc_h, ntc_v, semu),
            pltpu.async_copy(rtr_h, rtr_v, semu),
        ]
        for d in hdescs:
            d.wait()

        for j in range(B // _L):
            hi = hidx_v[pl.ds(j * _L, _L)]
            ri = ridx_v[pl.ds(j * _L, _L)]
            H_v[pl.ds(j * _L, _L)] = (plsc.load_gather(nth_v, [hi])
                                      + plsc.load_gather(rtq_v, [ri]))

        for d in tdescs:
            d.wait()

        def run_block(p, n, out_lo):
            buf_lo = p * C

            @plsc.parallel_loop(0, n, _L, unroll=16)
            def body(off):
                r = rc_v[0, pl.ds(buf_lo + off, _L)]
                c = rc_v[1, pl.ds(buf_lo + off, _L)]
                t = types_v[pl.ds(buf_lo + off, _L)]
                g = bids_v[pl.ds(buf_lo + off, _L)]
                out_v[pl.ds(out_lo + off, _L)] = (
                    plsc.load_gather(ntr_v, [r])
                    + plsc.load_gather(ntc_v, [c])
                    + plsc.load_gather(rtr_v, [t])
                    + plsc.load_gather(H_v, [g]))

        for ci in range(_NCHUNK):
            p = ci & 1
            nxt = pending if ci + 1 == _NCHUNK else start(
                base + (ci + 1) * C, C, 1 - p)
            for d in pending:
                d.wait()
            pending = nxt
            run_block(p, C, ci * C)

        pltpu.sync_copy(out_v, out_h.at[pl.ds(base, chunk)])

        if tail:
            @pl.when(wid == 0)
            def _():
                tail_lo = NW * chunk
                for d in start(tail_lo, tail, 0):
                    d.wait()
                run_block(0, tail, 0)
                pltpu.sync_copy(out_v.at[pl.ds(0, tail)],
                                out_h.at[pl.ds(tail_lo, tail)])

    return k(bei, types, bids, h_index, r_index,
             ntr, ntc, nth, rtr, rtq)


def kernel(batch_edge_index, batch_edge_type, batch_id, h_index, r_index,
           node_embeds, R, W, b):
    E = batch_edge_type.shape[0]
    ntr, ntc, nth, rtr, rtq = _tables_tc(node_embeds, R, W,
                                         b.astype(jnp.float32))
    out = _edge_sum_sc(
        batch_edge_index, batch_edge_type, batch_id,
        h_index, r_index, ntr, ntc, nth, rtr, rtq,
    )
    return out.reshape(E, 1)
